# baseline (speedup 1.0000x reference)
import functools

import jax
import jax.numpy as jnp
from jax import lax
from jax.experimental import pallas as pl
from jax.experimental.pallas import tpu as pltpu
from jax.experimental.pallas import tpu_sc as plsc

N = 16384
C = 1000
C0 = 504
NC = 2
NS = 16
NW = NC * NS
COLS_PER_W = N // NW
BLK = 128
NBLK = COLS_PER_W // BLK
L = 16
GRP = BLK // L


def _sc_one_hot_t(x_hbm, z_hbm, out_hbm, buf0, buf1, xv, semz, semx, sem0, sem1):
    wid = lax.axis_index("s") * NC + lax.axis_index("c")
    base = wid * COLS_PER_W

    cpz0 = pltpu.make_async_copy(z_hbm.at[pl.ds(0, C0)], buf0, semz)
    cpz0.start()
    cpz1 = pltpu.make_async_copy(z_hbm.at[pl.ds(C0, C - C0)], buf1, semz)
    cpz1.start()
    cpx = pltpu.make_async_copy(x_hbm.at[pl.ds(base, COLS_PER_W)], xv, semx)
    cpx.start()
    cpx.wait()
    cpz0.wait()
    cpz1.wait()

    lane = lax.iota(jnp.int32, L)
    ones = jnp.ones((L,), jnp.float32)
    zeros = jnp.zeros((L,), jnp.float32)

    def scatter_slab(b, vals):
        def group(g, _):
            rows = xv[pl.ds(b * BLK + g * L, L)]
            cols = g * L + lane
            plsc.store_scatter(buf0, [rows, cols], vals, mask=rows < C0)
            plsc.store_scatter(buf1, [rows - C0, cols], vals, mask=rows >= C0)
            return _

        lax.fori_loop(0, GRP, group, None)

    def dmas(b):
        cp0 = pltpu.make_async_copy(
            buf0,
            out_hbm.at[pl.ds(0, C0), pl.ds(base + b * BLK, BLK)],
            sem0,
        )
        cp1 = pltpu.make_async_copy(
            buf1,
            out_hbm.at[pl.ds(C0, C - C0), pl.ds(base + b * BLK, BLK)],
            sem1,
        )
        return cp0, cp1

    scatter_slab(0, ones)
    cp0, cp1 = dmas(0)
    cp0.start()
    cp1.start()

    def slab(b, _):
        p0, p1 = dmas(b - 1)
        p0.wait()
        p1.wait()
        scatter_slab(b - 1, zeros)
        scatter_slab(b, ones)
        c0, c1 = dmas(b)
        c0.start()
        c1.start()
        return _

    lax.fori_loop(1, NBLK, slab, None)
    l0, l1 = dmas(NBLK - 1)
    l0.wait()
    l1.wait()


@jax.jit
def kernel(x):
    mesh = plsc.VectorSubcoreMesh(core_axis_name="c", subcore_axis_name="s")
    call = functools.partial(
        pl.kernel,
        out_type=jax.ShapeDtypeStruct((C, N), jnp.float32),
        mesh=mesh,
        compiler_params=pltpu.CompilerParams(
            needs_layout_passes=False,
            skip_device_barrier=True,
            disable_bounds_checks=True,
            disable_semaphore_checks=True,
        ),
        scratch_types=[
            pltpu.VMEM((C0, BLK), jnp.float32),
            pltpu.VMEM((C - C0, BLK), jnp.float32),
            pltpu.VMEM((COLS_PER_W,), jnp.int32),
            pltpu.SemaphoreType.DMA,
            pltpu.SemaphoreType.DMA,
            pltpu.SemaphoreType.DMA,
            pltpu.SemaphoreType.DMA,
        ],
    )(_sc_one_hot_t)
    z = jnp.zeros((C, BLK), jnp.float32)
    out_t = call(x.astype(jnp.int32), z)
    return out_t.T


# speedup vs baseline: 1.0146x; 1.0146x over previous
import functools

import jax
import jax.numpy as jnp
from jax import lax
from jax.experimental import pallas as pl
from jax.experimental.pallas import tpu as pltpu
from jax.experimental.pallas import tpu_sc as plsc

N = 16384
C = 1000
NC = 2
NS = 16
NW = NC * NS
COLS_PER_W = N // NW
BLK = 128
NBLK = COLS_PER_W // BLK
L = 16
GRP = BLK // L


def _sc_one_hot_t(x_hbm, z_hbm, out_hbm, buf, xv, semz, semx, semo):
    wid = lax.axis_index("s") * NC + lax.axis_index("c")
    base = wid * COLS_PER_W

    cpz = pltpu.make_async_copy(z_hbm, buf, semz)
    cpz.start()
    cpx = pltpu.make_async_copy(x_hbm.at[pl.ds(base, COLS_PER_W)], xv, semx)
    cpx.start()
    cpx.wait()
    cpz.wait()

    lane = lax.iota(jnp.int32, L)
    ones = jnp.ones((L,), jnp.float32)
    zeros = jnp.zeros((L,), jnp.float32)

    def scatter_slab(b, vals):
        def group(g, _):
            rows = xv[pl.ds(b * BLK + g * L, L)]
            plsc.store_scatter(buf, [rows, g * L + lane], vals)
            return _

        lax.fori_loop(0, GRP, group, None)

    def slab(b, _):
        scatter_slab(b, ones)
        cp = pltpu.make_async_copy(
            buf, out_hbm.at[:, pl.ds(base + b * BLK, BLK)], semo
        )
        cp.start()
        cp.wait()
        scatter_slab(b, zeros)
        return _

    lax.fori_loop(0, NBLK, slab, None)


@jax.jit
def kernel(x):
    mesh = plsc.VectorSubcoreMesh(core_axis_name="c", subcore_axis_name="s")
    call = functools.partial(
        pl.kernel,
        out_type=jax.ShapeDtypeStruct((C, N), jnp.float32),
        mesh=mesh,
        compiler_params=pltpu.CompilerParams(
            needs_layout_passes=False,
            skip_device_barrier=True,
            disable_bounds_checks=True,
            disable_semaphore_checks=True,
        ),
        scratch_types=[
            pltpu.VMEM((C, BLK), jnp.float32),
            pltpu.VMEM((COLS_PER_W,), jnp.int32),
            pltpu.SemaphoreType.DMA,
            pltpu.SemaphoreType.DMA,
            pltpu.SemaphoreType.DMA,
        ],
    )(_sc_one_hot_t)
    z = jnp.zeros((C, BLK), jnp.float32)
    out_t = call(x.astype(jnp.int32), z)
    return out_t.T


# speedup vs baseline: 1.0239x; 1.0092x over previous
import functools

import jax
import jax.numpy as jnp
from jax import lax
from jax.experimental import pallas as pl
from jax.experimental.pallas import tpu as pltpu
from jax.experimental.pallas import tpu_sc as plsc

N = 16384
C = 1000
NC = 2
NS = 16
NW = NC * NS
COLS_PER_W = N // NW
BLK = 128
NBLK = COLS_PER_W // BLK
L = 16
GRP = BLK // L


def _sc_one_hot_t(x_hbm, z_hbm, out_hbm, buf, xv, semz, semx, semo):
    wid = lax.axis_index("s") * NC + lax.axis_index("c")
    base = wid * COLS_PER_W

    cpz = pltpu.make_async_copy(z_hbm, buf, semz)
    cpz.start()
    cpx = pltpu.make_async_copy(x_hbm.at[pl.ds(base, COLS_PER_W)], xv, semx)
    cpx.start()
    cpx.wait()
    cpz.wait()

    lane = lax.iota(jnp.int32, L)
    ones = jnp.ones((L,), jnp.float32)
    zeros = jnp.zeros((L,), jnp.float32)

    def scatter_slab(b, vals):
        def group(g, _):
            rows = xv[pl.ds(b * BLK + g * L, L)]
            plsc.store_scatter(buf, [rows, g * L + lane], vals)
            return _

        lax.fori_loop(0, GRP, group, None)

    def slab(b, _):
        scatter_slab(b, ones)
        cp = pltpu.make_async_copy(
            buf, out_hbm.at[:, pl.ds(base + b * BLK, BLK)], semo
        )
        cp.start()
        cp.wait()
        scatter_slab(b, zeros)
        return _

    lax.fori_loop(0, NBLK, slab, None)


@jax.jit
def kernel(x):
    mesh = plsc.VectorSubcoreMesh(core_axis_name="c", subcore_axis_name="s")
    call = functools.partial(
        pl.kernel,
        out_type=jax.ShapeDtypeStruct((C, N), jnp.float32),
        mesh=mesh,
        compiler_params=pltpu.CompilerParams(needs_layout_passes=False),
        scratch_types=[
            pltpu.VMEM((C, BLK), jnp.float32),
            pltpu.VMEM((COLS_PER_W,), jnp.int32),
            pltpu.SemaphoreType.DMA,
            pltpu.SemaphoreType.DMA,
            pltpu.SemaphoreType.DMA,
        ],
    )(_sc_one_hot_t)
    z = jnp.zeros((C, BLK), jnp.float32)
    out_t = call(x.astype(jnp.int32), z)
    return out_t.T
